# baseline (speedup 1.0000x reference)
import functools

import jax
import jax.numpy as jnp
from jax import lax
from jax.experimental import pallas as pl
from jax.experimental.pallas import tpu as pltpu
from jax.experimental.pallas import tpu_sc as plsc


def _ae_body(x_ref, enc_ref, dec_ref, rep_ref, recon_ref):
    rep = jnp.tanh(jnp.dot(x_ref[...], enc_ref[...], preferred_element_type=jnp.float32))
    rep_ref[...] = rep
    recon_ref[...] = jnp.dot(rep, dec_ref[...], preferred_element_type=jnp.float32)


def _z_body(bd_ref, repg_ref, head_ref, zg_ref):
    zg_ref[...] = jnp.dot(repg_ref[...], head_ref[0], preferred_element_type=jnp.float32)


def kernel(x_scaled, digits, W_enc, W_dec, heads):
    B, D_IN = x_scaled.shape
    REP = W_enc.shape[1]
    K, _, ZD = heads.shape
    BT = 1024
    nb = B // BT

    T = 256
    NP = B + K * T
    NBZ = NP // T
    NB48 = 48
    NW = 32
    BPW = NP // NW
    NCH = B // 16

    rep, recon = pl.pallas_call(
        _ae_body,
        grid=(nb,),
        in_specs=[
            pl.BlockSpec((BT, D_IN), lambda i: (i, 0)),
            pl.BlockSpec((D_IN, REP), lambda i: (0, 0)),
            pl.BlockSpec((REP, D_IN), lambda i: (0, 0)),
        ],
        out_specs=[
            pl.BlockSpec((BT, REP), lambda i: (i, 0)),
            pl.BlockSpec((BT, D_IN), lambda i: (i, 0)),
        ],
        out_shape=[
            jax.ShapeDtypeStruct((B, REP), jnp.float32),
            jax.ShapeDtypeStruct((B, D_IN), jnp.float32),
        ],
        compiler_params=pltpu.CompilerParams(
            dimension_semantics=("arbitrary",),
            vmem_limit_bytes=100 * 1024 * 1024,
        ),
    )(x_scaled, W_enc, W_dec)

    mesh = plsc.VectorSubcoreMesh(core_axis_name="c", subcore_axis_name="s")

    @functools.partial(
        pl.kernel,
        mesh=mesh,
        compiler_params=pltpu.CompilerParams(needs_layout_passes=False),
        out_type=[
            jax.ShapeDtypeStruct((NP,), jnp.int32),
            jax.ShapeDtypeStruct((NB48,), jnp.int32),
            jax.ShapeDtypeStruct((NP, REP), jnp.float32),
        ],
        scratch_types=[
            pltpu.VMEM((B,), jnp.int32),
            pltpu.VMEM((B + T,), jnp.int32),
            pltpu.VMEM((16,), jnp.int32),
            pltpu.VMEM((NB48,), jnp.int32),
            pltpu.VMEM((16,), jnp.int32),
            pltpu.VMEM((16, REP), jnp.float32),
            pltpu.VMEM_SHARED((16,), jnp.int32),
            pltpu.VMEM_SHARED((NB48,), jnp.int32),
            pltpu.VMEM_SHARED((NP,), jnp.int32),
            pltpu.SemaphoreType.DMA,
        ],
    )
    def route_gather(dig_hbm, rep_hbm, perm_hbm, blkdig_hbm, repg_hbm,
                     dig_v, comp_v, st_v, blk_v, idx_v, rows_v,
                     counts_sh, blkdig_sh, perm_sh, gsem):
        cid = lax.axis_index("c")
        sid = lax.axis_index("s")
        wid = sid * 2 + cid
        lanes = jnp.arange(16, dtype=jnp.int32)
        al = lambda v: pl.multiple_of(v, 16)

        @pl.when(sid == 0)
        def _zero_shared():
            st_v[...] = jnp.zeros((16,), jnp.int32)
            pltpu.sync_copy(st_v, counts_sh)
            for j in range(NB48 // 16):
                pltpu.sync_copy(st_v, blkdig_sh.at[pl.ds(16 * j, 16)])

        @pl.when(sid < K)
        def _count():
            pltpu.sync_copy(dig_hbm, dig_v)
            def body(c, acc):
                m = (dig_v[pl.ds(al(c * 16), 16)] == sid)
                return acc + jnp.sum(m.astype(jnp.int32))
            cnt = lax.fori_loop(0, NCH, body, 0)
            st_v[...] = jnp.where(lanes == sid, cnt, 0)

        plsc.subcore_barrier()

        @pl.when(sid < K)
        def _pub_counts():
            pltpu.sync_copy(st_v, counts_sh.at[lanes], add=True)

        plsc.subcore_barrier()

        pltpu.sync_copy(counts_sh, st_v)
        counts = st_v[...]
        pc = ((counts + (T - 1)) // T) * T
        incl = plsc.cumsum(pc)
        excl = incl - pc
        total_used = jnp.sum(jnp.where(lanes == (K - 1), incl, 0))
        astart = jnp.sum(jnp.where(lanes == sid, excl, 0))
        mypc = jnp.sum(jnp.where(lanes == sid, pc, 0))
        nblk = mypc // T

        @pl.when(sid < K)
        def _fill():
            sent = jnp.full((16,), B, jnp.int32)
            def zb(c, _):
                comp_v[pl.ds(al(c * 16), 16)] = sent
                return 0
            lax.fori_loop(0, (B + T) // 16, zb, 0)

            def fb(c, off):
                m = (dig_v[pl.ds(al(c * 16), 16)] == sid)
                m32 = m.astype(jnp.int32)
                rank = off + plsc.cumsum(m32) - 1
                plsc.store_scatter(comp_v, [rank], lanes + c * 16, mask=m)
                return off + jnp.sum(m32)
            lax.fori_loop(0, NCH, fb, 0)

            def db(j, _):
                src = comp_v.at[pl.ds(al(j * 16), 16)]
                pltpu.sync_copy(src, perm_sh.at[pl.ds(al(astart + j * 16), 16)])

                @pl.when(cid == 0)
                def _hbm():
                    pltpu.sync_copy(src, perm_hbm.at[pl.ds(al(astart + j * 16), 16)])
                return 0
            lax.fori_loop(0, mypc // 16, db, 0)

            for j in range(NB48 // 16):
                blk_v[pl.ds(16 * j, 16)] = jnp.zeros((16,), jnp.int32)
            bb = astart // T
            for j in range(2):
                lj = lanes + 16 * j
                plsc.store_scatter(blk_v, [bb + lj],
                                   jnp.full((16,), sid, jnp.int32),
                                   mask=lj < nblk)
            for j in range(NB48 // 16):
                pltpu.sync_copy(blk_v.at[pl.ds(16 * j, 16)],
                                blkdig_sh.at[lanes + 16 * j], add=True)

        @pl.when(sid == K)
        def _tail():
            st_v[...] = jnp.full((16,), B, jnp.int32)
            def tb(j, _):
                pltpu.sync_copy(st_v, perm_sh.at[pl.ds(al(total_used + j * 16), 16)])

                @pl.when(cid == 0)
                def _hbm():
                    pltpu.sync_copy(st_v, perm_hbm.at[pl.ds(al(total_used + j * 16), 16)])
                return 0
            lax.fori_loop(0, (NP - total_used) // 16, tb, 0)

        plsc.subcore_barrier()

        @pl.when(wid == 0)
        def _pub_blkdig():
            pltpu.sync_copy(blkdig_sh, blkdig_hbm)

        base = wid * BPW
        def gb(i, _):
            pltpu.sync_copy(perm_sh.at[pl.ds(al(base + i * 16), 16)], idx_v)
            idx_v[...] = jnp.minimum(idx_v[...], B - 1)
            pltpu.async_copy(rep_hbm.at[idx_v], rows_v, gsem).wait()
            pltpu.sync_copy(rows_v, repg_hbm.at[pl.ds(al(base + i * 16), 16)])
            return 0
        lax.fori_loop(0, BPW // 16, gb, 0)

    perm, blkdig, repg = route_gather(digits, rep)

    grid_spec = pltpu.PrefetchScalarGridSpec(
        num_scalar_prefetch=1,
        grid=(NBZ,),
        in_specs=[
            pl.BlockSpec((T, REP), lambda i, bd: (i, 0)),
            pl.BlockSpec((1, REP, ZD), lambda i, bd: (bd[i], 0, 0)),
        ],
        out_specs=pl.BlockSpec((T, ZD), lambda i, bd: (i, 0)),
    )
    zg = pl.pallas_call(
        _z_body,
        grid_spec=grid_spec,
        out_shape=jax.ShapeDtypeStruct((NP, ZD), jnp.float32),
    )(blkdig[:NBZ], repg, heads)

    @functools.partial(
        pl.kernel,
        mesh=mesh,
        compiler_params=pltpu.CompilerParams(needs_layout_passes=False),
        out_type=jax.ShapeDtypeStruct((B + 16, ZD), jnp.float32),
        scratch_types=[
            pltpu.VMEM((16,), jnp.int32),
            pltpu.VMEM((16, ZD), jnp.float32),
            pltpu.SemaphoreType.DMA,
        ],
    )
    def unscatter(perm_hbm, zg_hbm, zout_hbm, idx_v, rows_v, ssem):
        cid = lax.axis_index("c")
        sid = lax.axis_index("s")
        wid = sid * 2 + cid
        base = wid * BPW
        al = lambda v: pl.multiple_of(v, 16)
        def sb(i, _):
            pltpu.sync_copy(perm_hbm.at[pl.ds(al(base + i * 16), 16)], idx_v)
            pltpu.sync_copy(zg_hbm.at[pl.ds(al(base + i * 16), 16)], rows_v)
            pltpu.async_copy(rows_v, zout_hbm.at[idx_v], ssem).wait()
            return 0
        lax.fori_loop(0, BPW // 16, sb, 0)

    zout = unscatter(perm, zg)
    return rep, recon, zout[:B]


# speedup vs baseline: 7.8656x; 7.8656x over previous
import jax
import jax.numpy as jnp
from jax.experimental import pallas as pl
from jax.experimental.pallas import tpu as pltpu


def _body(dig_ref, x_ref, enc_ref, dec_ref, headsw_ref, rep_ref, recon_ref, z_ref):
    ZD = z_ref.shape[1]
    K = headsw_ref.shape[1] // ZD
    rep = jnp.tanh(jnp.dot(x_ref[...], enc_ref[...], preferred_element_type=jnp.float32))
    rep_ref[...] = rep
    recon_ref[...] = jnp.dot(rep, dec_ref[...], preferred_element_type=jnp.float32)
    zw = jnp.dot(rep, headsw_ref[...], preferred_element_type=jnp.float32)
    dig = dig_ref[...]
    acc = zw[:, 0:ZD]
    for k in range(1, K):
        acc = jnp.where(dig == k, zw[:, k * ZD:(k + 1) * ZD], acc)
    z_ref[...] = acc


def kernel(x_scaled, digits, W_enc, W_dec, heads):
    B, D_IN = x_scaled.shape
    REP = W_enc.shape[1]
    K, _, ZD = heads.shape
    BT = 1024
    nb = B // BT
    dig2 = digits.reshape(B, 1)
    heads_wide = heads.transpose(1, 0, 2).reshape(REP, K * ZD)

    rep, recon, z = pl.pallas_call(
        _body,
        grid=(nb,),
        in_specs=[
            pl.BlockSpec((BT, 1), lambda i: (i, 0)),
            pl.BlockSpec((BT, D_IN), lambda i: (i, 0)),
            pl.BlockSpec((D_IN, REP), lambda i: (0, 0)),
            pl.BlockSpec((REP, D_IN), lambda i: (0, 0)),
            pl.BlockSpec((REP, K * ZD), lambda i: (0, 0)),
        ],
        out_specs=[
            pl.BlockSpec((BT, REP), lambda i: (i, 0)),
            pl.BlockSpec((BT, D_IN), lambda i: (i, 0)),
            pl.BlockSpec((BT, ZD), lambda i: (i, 0)),
        ],
        out_shape=[
            jax.ShapeDtypeStruct((B, REP), jnp.float32),
            jax.ShapeDtypeStruct((B, D_IN), jnp.float32),
            jax.ShapeDtypeStruct((B, ZD), jnp.float32),
        ],
        compiler_params=pltpu.CompilerParams(
            dimension_semantics=("arbitrary",),
            vmem_limit_bytes=100 * 1024 * 1024,
        ),
    )(dig2, x_scaled, W_enc, W_dec, heads_wide)
    return rep, recon, z
